# initial kernel scaffold (unmeasured)
import jax
import jax.numpy as jnp
from jax import lax
from jax.experimental import pallas as pl
from jax.experimental.pallas import tpu as pltpu

N_DEV = 4
E_LOCAL = 2


def kernel(x, router_W, route_idx, expert_W):
    n_tok, d = x.shape
    h = expert_W.shape[-1]
    n_exp = router_W.shape[-1]

    def body(x_ref, rw_ref, idx_ref, ew_ref, out_ref, comm_ref,
             send_sems, recv_sems):
        my = lax.axis_index("i")
        right = lax.rem(my + 1, N_DEV)

        xf = x_ref[:, :]
        scores = jnp.dot(xf, rw_ref[:, :], preferred_element_type=jnp.float32)
        smax = jnp.max(scores, axis=1, keepdims=True)
        p = jnp.exp(scores - smax)
        probs = p / jnp.sum(p, axis=1, keepdims=True)

        iota = lax.broadcasted_iota(jnp.int32, (n_tok, n_exp), 1)
        idx0 = idx_ref[:, 0:1]
        idx1 = idx_ref[:, 1:2]
        g0 = jnp.sum(jnp.where(iota == idx0, probs, 0.0), axis=1, keepdims=True)
        g1 = jnp.sum(jnp.where(iota == idx1, probs, 0.0), axis=1, keepdims=True)
        gs = g0 + g1
        w0 = g0 / gs
        w1 = g1 / gs

        acc = jnp.zeros((n_tok, h), jnp.float32)
        for le in range(E_LOCAL):
            gid = my * E_LOCAL + le
            gate = (jnp.where(idx0 == gid, w0, 0.0)
                    + jnp.where(idx1 == gid, w1, 0.0))
            xg = (xf * gate).astype(jnp.bfloat16)
            we = ew_ref[le, :, :].astype(jnp.bfloat16)
            acc = acc + jnp.dot(xg, we, preferred_element_type=jnp.float32)

        comm_ref[0, :, :] = acc

        for hop in range(N_DEV - 1):
            rdma = pltpu.make_async_remote_copy(
                src_ref=comm_ref.at[hop],
                dst_ref=comm_ref.at[hop + 1],
                send_sem=send_sems.at[hop],
                recv_sem=recv_sems.at[hop],
                device_id=(right,),
                device_id_type=pl.DeviceIdType.MESH,
            )
            rdma.start()
            rdma.wait()
            acc = acc + comm_ref[hop + 1, :, :]

        out_ref[:, :] = acc

    return pl.pallas_call(
        body,
        out_shape=jax.ShapeDtypeStruct((n_tok, h), jnp.float32),
        in_specs=[pl.BlockSpec(memory_space=pltpu.VMEM)] * 4,
        out_specs=pl.BlockSpec(memory_space=pltpu.VMEM),
        scratch_shapes=[
            pltpu.VMEM((N_DEV, n_tok, h), jnp.float32),
            pltpu.SemaphoreType.DMA((N_DEV - 1,)),
            pltpu.SemaphoreType.DMA((N_DEV - 1,)),
        ],
        compiler_params=pltpu.CompilerParams(collective_id=0),
    )(x, router_W, route_idx, expert_W)


# baseline (device time: 22455 ns/iter reference)
import jax
import jax.numpy as jnp
from jax import lax
from jax.experimental import pallas as pl
from jax.experimental.pallas import tpu as pltpu

N_DEV = 4
E_LOCAL = 2


def kernel(x, router_W, route_idx, expert_W):
    n_tok, d = x.shape
    h = expert_W.shape[-1]
    n_exp = router_W.shape[-1]

    def body(x_ref, rw_ref, idx_ref, ew_ref, out_ref, comm_ref,
             send_sems, recv_sems):
        my = lax.axis_index("i")
        right = lax.rem(my + 1, N_DEV)

        xf = x_ref[:, :]
        scores = jnp.dot(xf, rw_ref[:, :], preferred_element_type=jnp.float32)
        smax = jnp.max(scores, axis=1, keepdims=True)
        p = jnp.exp(scores - smax)
        probs = p / jnp.sum(p, axis=1, keepdims=True)

        iota = lax.broadcasted_iota(jnp.int32, (n_tok, n_exp), 1)
        idx0 = idx_ref[:, 0:1]
        idx1 = idx_ref[:, 1:2]
        g0 = jnp.sum(jnp.where(iota == idx0, probs, 0.0), axis=1, keepdims=True)
        g1 = jnp.sum(jnp.where(iota == idx1, probs, 0.0), axis=1, keepdims=True)
        gs = g0 + g1
        w0 = g0 / gs
        w1 = g1 / gs

        acc = jnp.zeros((n_tok, h), jnp.float32)
        for le in range(E_LOCAL):
            gid = my * E_LOCAL + le
            gate = (jnp.where(idx0 == gid, w0, 0.0)
                    + jnp.where(idx1 == gid, w1, 0.0))
            xg = (xf * gate).astype(jnp.bfloat16)
            we = ew_ref[le, :, :].astype(jnp.bfloat16)
            acc = acc + jnp.dot(xg, we, preferred_element_type=jnp.float32)

        comm_ref[0, :, :] = acc

        for hop in range(N_DEV - 1):
            rdma = pltpu.make_async_remote_copy(
                src_ref=comm_ref.at[hop],
                dst_ref=comm_ref.at[hop + 1],
                send_sem=send_sems.at[hop],
                recv_sem=recv_sems.at[hop],
                device_id=(right,),
                device_id_type=pl.DeviceIdType.MESH,
            )
            rdma.start()
            rdma.wait()
            acc = acc + comm_ref[hop + 1, :, :]

        out_ref[:, :] = acc

    return pl.pallas_call(
        body,
        out_shape=jax.ShapeDtypeStruct((n_tok, h), jnp.float32),
        in_specs=[pl.BlockSpec(memory_space=pltpu.VMEM)] * 4,
        out_specs=pl.BlockSpec(memory_space=pltpu.VMEM),
        scratch_shapes=[
            pltpu.VMEM((N_DEV, n_tok, h), jnp.float32),
            pltpu.SemaphoreType.DMA((N_DEV - 1,)),
            pltpu.SemaphoreType.DMA((N_DEV - 1,)),
        ],
    )(x, router_W, route_idx, expert_W)


# device time: 12561 ns/iter; 1.7877x vs baseline; 1.7877x over previous
import functools

import jax
import jax.numpy as jnp
from jax import lax
from jax.experimental import pallas as pl
from jax.experimental.pallas import tpu as pltpu

N_DEV = 4
E_LOCAL = 2


def kernel(x, router_W, route_idx, expert_W):
    n_tok, d = x.shape
    h = expert_W.shape[-1]
    n_exp = router_W.shape[-1]

    def body(x_ref, rw_ref, idx_ref, ew_ref, out_ref, comm_ref,
             send_sems, recv_sems):
        me = lax.axis_index("i")
        peers = [lax.rem(me + off, N_DEV) for off in range(1, N_DEV)]

        barrier = pltpu.get_barrier_semaphore()
        for p in peers:
            pl.semaphore_signal(barrier, inc=1, device_id=(p,),
                                device_id_type=pl.DeviceIdType.MESH)
        pl.semaphore_wait(barrier, N_DEV - 1)

        xf = x_ref[:, :]
        scores = jnp.dot(xf, rw_ref[:, :], preferred_element_type=jnp.float32)
        smax = jnp.max(scores, axis=1, keepdims=True)
        p_ = jnp.exp(scores - smax)
        probs = p_ / jnp.sum(p_, axis=1, keepdims=True)

        iota = lax.broadcasted_iota(jnp.int32, (n_tok, n_exp), 1)
        idx0 = idx_ref[:, 0:1]
        idx1 = idx_ref[:, 1:2]
        g0 = jnp.sum(jnp.where(iota == idx0, probs, 0.0), axis=1, keepdims=True)
        g1 = jnp.sum(jnp.where(iota == idx1, probs, 0.0), axis=1, keepdims=True)
        gs = g0 + g1
        w0 = g0 / gs
        w1 = g1 / gs

        acc = jnp.zeros((n_tok, h), jnp.float32)
        for le in range(E_LOCAL):
            gid = me * E_LOCAL + le
            gate = (jnp.where(idx0 == gid, w0, 0.0)
                    + jnp.where(idx1 == gid, w1, 0.0))
            xg = (xf * gate).astype(jnp.bfloat16)
            we = ew_ref[le, :, :].astype(jnp.bfloat16)
            acc = acc + jnp.dot(xg, we, preferred_element_type=jnp.float32)

        comm_ref[me, :, :] = acc.astype(jnp.bfloat16)

        sends = []
        for k, p in enumerate(peers):
            rdma = pltpu.make_async_remote_copy(
                src_ref=comm_ref.at[me],
                dst_ref=comm_ref.at[me],
                send_sem=send_sems.at[k],
                recv_sem=recv_sems.at[me],
                device_id=(p,),
                device_id_type=pl.DeviceIdType.MESH,
            )
            rdma.start()
            sends.append(rdma)

        for p in peers:
            recv = pltpu.make_async_remote_copy(
                src_ref=comm_ref.at[me],
                dst_ref=comm_ref.at[p],
                send_sem=send_sems.at[0],
                recv_sem=recv_sems.at[p],
                device_id=(p,),
                device_id_type=pl.DeviceIdType.MESH,
            )
            recv.wait_recv()
            acc = acc + comm_ref[p, :, :].astype(jnp.float32)

        out_ref[:, :] = acc

        for rdma in sends:
            rdma.wait_send()

        @functools.partial(pl.run_scoped, exit_sem=pltpu.SemaphoreType.REGULAR)
        def _(exit_sem):
            for p in peers:
                pl.semaphore_signal(exit_sem, inc=1, device_id=(p,),
                                    device_id_type=pl.DeviceIdType.MESH)
            pl.semaphore_wait(exit_sem, N_DEV - 1)

    return pl.pallas_call(
        body,
        out_shape=jax.ShapeDtypeStruct((n_tok, h), jnp.float32),
        in_specs=[pl.BlockSpec(memory_space=pltpu.VMEM)] * 4,
        out_specs=pl.BlockSpec(memory_space=pltpu.VMEM),
        scratch_shapes=[
            pltpu.VMEM((N_DEV, n_tok, h), jnp.bfloat16),
            pltpu.SemaphoreType.DMA((N_DEV - 1,)),
            pltpu.SemaphoreType.DMA((N_DEV,)),
        ],
        compiler_params=pltpu.CompilerParams(collective_id=0),
    )(x, router_W, route_idx, expert_W)


# device time: 10505 ns/iter; 2.1376x vs baseline; 1.1957x over previous
import jax
import jax.numpy as jnp
from jax import lax
from jax.experimental import pallas as pl
from jax.experimental.pallas import tpu as pltpu

N_DEV = 4
E_LOCAL = 2


def kernel(x, router_W, route_idx, expert_W):
    n_tok, d = x.shape
    h = expert_W.shape[-1]
    n_exp = router_W.shape[-1]

    def body(x_ref, rw_ref, idx_ref, ew_ref, out_ref, comm_ref,
             send_sems, recv_sems):
        me = lax.axis_index("i")
        peers = [lax.rem(me + off, N_DEV) for off in range(1, N_DEV)]

        barrier = pltpu.get_barrier_semaphore()
        for p in peers:
            pl.semaphore_signal(barrier, inc=1, device_id=(p,),
                                device_id_type=pl.DeviceIdType.MESH)

        xf = x_ref[:, :]
        scores = jnp.dot(xf, rw_ref[:, :], preferred_element_type=jnp.float32)
        smax = jnp.max(scores, axis=1, keepdims=True)
        p_ = jnp.exp(scores - smax)
        probs = p_ / jnp.sum(p_, axis=1, keepdims=True)

        iota = lax.broadcasted_iota(jnp.int32, (n_tok, n_exp), 1)
        idx0 = idx_ref[:, 0:1]
        idx1 = idx_ref[:, 1:2]
        g0 = jnp.sum(jnp.where(iota == idx0, probs, 0.0), axis=1, keepdims=True)
        g1 = jnp.sum(jnp.where(iota == idx1, probs, 0.0), axis=1, keepdims=True)
        gs = g0 + g1
        w0 = g0 / gs
        w1 = g1 / gs

        acc = jnp.zeros((n_tok, h), jnp.float32)
        for le in range(E_LOCAL):
            gid = me * E_LOCAL + le
            gate = (jnp.where(idx0 == gid, w0, 0.0)
                    + jnp.where(idx1 == gid, w1, 0.0))
            xg = (xf * gate).astype(jnp.bfloat16)
            we = ew_ref[le, :, :].astype(jnp.bfloat16)
            acc = acc + jnp.dot(xg, we, preferred_element_type=jnp.float32)

        comm_ref[me, :, :] = acc.astype(jnp.bfloat16)

        pl.semaphore_wait(barrier, N_DEV - 1)

        send_order = [1, 0, 2]
        sends = []
        for k in send_order:
            p = peers[k]
            rdma = pltpu.make_async_remote_copy(
                src_ref=comm_ref.at[me],
                dst_ref=comm_ref.at[me],
                send_sem=send_sems.at[k],
                recv_sem=recv_sems.at[me],
                device_id=(p,),
                device_id_type=pl.DeviceIdType.MESH,
            )
            rdma.start()
            sends.append(rdma)

        for k in (0, 2, 1):
            p = peers[k]
            recv = pltpu.make_async_remote_copy(
                src_ref=comm_ref.at[me],
                dst_ref=comm_ref.at[p],
                send_sem=send_sems.at[0],
                recv_sem=recv_sems.at[p],
                device_id=(p,),
                device_id_type=pl.DeviceIdType.MESH,
            )
            recv.wait_recv()
            acc = acc + comm_ref[p, :, :].astype(jnp.float32)

        out_ref[:, :] = acc

        for rdma in sends:
            rdma.wait_send()

    return pl.pallas_call(
        body,
        out_shape=jax.ShapeDtypeStruct((n_tok, h), jnp.float32),
        in_specs=[pl.BlockSpec(memory_space=pltpu.VMEM)] * 4,
        out_specs=pl.BlockSpec(memory_space=pltpu.VMEM),
        scratch_shapes=[
            pltpu.VMEM((N_DEV, n_tok, h), jnp.bfloat16),
            pltpu.SemaphoreType.DMA((N_DEV - 1,)),
            pltpu.SemaphoreType.DMA((N_DEV,)),
        ],
        compiler_params=pltpu.CompilerParams(collective_id=0),
    )(x, router_W, route_idx, expert_W)


# device time: 10417 ns/iter; 2.1556x vs baseline; 1.0084x over previous
import jax
import jax.numpy as jnp
from jax import lax
from jax.experimental import pallas as pl
from jax.experimental.pallas import tpu as pltpu

N_DEV = 4
E_LOCAL = 2


def kernel(x, router_W, route_idx, expert_W):
    n_tok, d = x.shape
    h = expert_W.shape[-1]
    n_exp = router_W.shape[-1]

    def body(x_ref, rw_ref, idx_ref, ew_ref, out_ref, comm_ref,
             send_sems, recv_sems):
        me = lax.axis_index("i")
        peers = [lax.rem(me + off, N_DEV) for off in range(1, N_DEV)]

        barrier = pltpu.get_barrier_semaphore()
        for p in peers:
            pl.semaphore_signal(barrier, inc=1, device_id=(p,),
                                device_id_type=pl.DeviceIdType.MESH)

        xf = x_ref[:, :]
        scores = jnp.dot(xf, rw_ref[:, :], preferred_element_type=jnp.float32)
        smax = jnp.max(scores, axis=1, keepdims=True)
        p_ = jnp.exp(scores - smax)
        probs = p_ / jnp.sum(p_, axis=1, keepdims=True)

        iota = lax.broadcasted_iota(jnp.int32, (n_tok, n_exp), 1)
        idx0 = idx_ref[:, 0:1]
        idx1 = idx_ref[:, 1:2]
        g0 = jnp.sum(jnp.where(iota == idx0, probs, 0.0), axis=1, keepdims=True)
        g1 = jnp.sum(jnp.where(iota == idx1, probs, 0.0), axis=1, keepdims=True)
        gs = g0 + g1
        w0 = g0 / gs
        w1 = g1 / gs

        gated = []
        for le in range(E_LOCAL):
            gid = me * E_LOCAL + le
            gate = (jnp.where(idx0 == gid, w0, 0.0)
                    + jnp.where(idx1 == gid, w1, 0.0))
            gated.append((xf * gate).astype(jnp.bfloat16))
        xg = jnp.concatenate(gated, axis=1)
        wcat = ew_ref[:, :, :].reshape(E_LOCAL * d, h).astype(jnp.bfloat16)
        acc = jnp.dot(xg, wcat, preferred_element_type=jnp.float32)

        comm_ref[me, :, :] = acc.astype(jnp.bfloat16)

        pl.semaphore_wait(barrier, N_DEV - 1)

        send_order = [1, 0, 2]
        sends = []
        for k in send_order:
            p = peers[k]
            rdma = pltpu.make_async_remote_copy(
                src_ref=comm_ref.at[me],
                dst_ref=comm_ref.at[me],
                send_sem=send_sems.at[k],
                recv_sem=recv_sems.at[me],
                device_id=(p,),
                device_id_type=pl.DeviceIdType.MESH,
            )
            rdma.start()
            sends.append(rdma)

        for k in (0, 2, 1):
            p = peers[k]
            recv = pltpu.make_async_remote_copy(
                src_ref=comm_ref.at[me],
                dst_ref=comm_ref.at[p],
                send_sem=send_sems.at[0],
                recv_sem=recv_sems.at[p],
                device_id=(p,),
                device_id_type=pl.DeviceIdType.MESH,
            )
            recv.wait_recv()
            acc = acc + comm_ref[p, :, :].astype(jnp.float32)

        out_ref[:, :] = acc

        for rdma in sends:
            rdma.wait_send()

    return pl.pallas_call(
        body,
        out_shape=jax.ShapeDtypeStruct((n_tok, h), jnp.float32),
        in_specs=[pl.BlockSpec(memory_space=pltpu.VMEM)] * 4,
        out_specs=pl.BlockSpec(memory_space=pltpu.VMEM),
        scratch_shapes=[
            pltpu.VMEM((N_DEV, n_tok, h), jnp.bfloat16),
            pltpu.SemaphoreType.DMA((N_DEV - 1,)),
            pltpu.SemaphoreType.DMA((N_DEV,)),
        ],
        compiler_params=pltpu.CompilerParams(collective_id=0),
    )(x, router_W, route_idx, expert_W)
